# Initial kernel scaffold; baseline (speedup 1.0000x reference)
import jax
import jax.numpy as jnp
from jax.experimental import pallas as pl


def kernel(input_seq, seq_lengths, supports, Wg0, bg0, Wc0, bc0, Wg1, bg1, Wc1, bc1, Wfc, bfc):
    raise NotImplementedError("write your pallas kernel here")


# speedup vs baseline: 14.0782x; 14.0782x over previous
import jax
import jax.numpy as jnp
from jax.experimental import pallas as pl
from jax.experimental.pallas import tpu as pltpu

N = 512
D = 128
H = 128
T = 16
B = 4
K = 3
C = 4
F32 = jnp.float32


def _dot(a, b):
    return jnp.dot(a, b, preferred_element_type=F32)


def _pre_body(x_ref, s_ref, w_ref, b_ref, a_ref):
    x0 = x_ref[0]
    s = s_ref[...]
    x1 = _dot(s, x0)
    x2 = 2.0 * _dot(s, x1) - x0
    w = w_ref[...]
    bias = b_ref[0]
    for bi in range(B):
        sl = slice(bi * D, (bi + 1) * D)
        xc = jnp.concatenate([x0[:, sl], x1[:, sl], x2[:, sl]], axis=1)
        a_ref[0, bi] = _dot(xc, w) + bias


def _precompute(x, s, w_in, bias_in):
    return pl.pallas_call(
        _pre_body,
        grid=(T,),
        in_specs=[
            pl.BlockSpec((1, N, B * D), lambda t: (t, 0, 0)),
            pl.BlockSpec((N, N), lambda t: (0, 0)),
            pl.BlockSpec((K * D, 3 * H), lambda t: (0, 0)),
            pl.BlockSpec((1, 3 * H), lambda t: (0, 0)),
        ],
        out_specs=pl.BlockSpec((1, B, N, 3 * H), lambda t: (t, 0, 0, 0)),
        out_shape=jax.ShapeDtypeStruct((T, B, N, 3 * H), F32),
    )(x, s, w_in, bias_in)


def _gru_step(a_ref, s, wg, wc, state_ref, rs_ref):
    h0 = state_ref[...]
    h1 = _dot(s, h0)
    h2 = 2.0 * _dot(s, h1) - h0
    us = []
    for bi in range(B):
        sl = slice(bi * H, (bi + 1) * H)
        xc = jnp.concatenate([h0[:, sl], h1[:, sl], h2[:, sl]], axis=1)
        g = jax.nn.sigmoid(a_ref[0, bi, :, : 2 * H] + _dot(xc, wg))
        r, u = g[:, :H], g[:, H:]
        rs_ref[:, sl] = r * h0[:, sl]
        us.append(u)
    rs0 = rs_ref[...]
    rs1 = _dot(s, rs0)
    rs2 = 2.0 * _dot(s, rs1) - rs0
    new_states = []
    for bi in range(B):
        sl = slice(bi * H, (bi + 1) * H)
        xc = jnp.concatenate([rs0[:, sl], rs1[:, sl], rs2[:, sl]], axis=1)
        c = jnp.tanh(a_ref[0, bi, :, 2 * H:] + _dot(xc, wc))
        u = us[bi]
        new_states.append(u * h0[:, sl] + (1.0 - u) * c)
    return new_states


def _rec_body(a_ref, s_ref, wg_ref, wc_ref, o_ref, state_ref, rs_ref):
    t = pl.program_id(0)

    @pl.when(t == 0)
    def _():
        state_ref[...] = jnp.zeros_like(state_ref)

    new_states = _gru_step(a_ref, s_ref[...], wg_ref[...], wc_ref[...],
                           state_ref, rs_ref)
    for bi in range(B):
        sl = slice(bi * H, (bi + 1) * H)
        state_ref[:, sl] = new_states[bi]
        o_ref[0, :, sl] = new_states[bi]


def _recurrence(a, s, wg_h, wc_h):
    return pl.pallas_call(
        _rec_body,
        grid=(T,),
        in_specs=[
            pl.BlockSpec((1, B, N, 3 * H), lambda t: (t, 0, 0, 0)),
            pl.BlockSpec((N, N), lambda t: (0, 0)),
            pl.BlockSpec((K * H, 2 * H), lambda t: (0, 0)),
            pl.BlockSpec((K * H, H), lambda t: (0, 0)),
        ],
        out_specs=pl.BlockSpec((1, N, B * H), lambda t: (t, 0, 0)),
        out_shape=jax.ShapeDtypeStruct((T, N, B * H), F32),
        scratch_shapes=[
            pltpu.VMEM((N, B * H), F32),
            pltpu.VMEM((N, B * H), F32),
        ],
    )(a, s, wg_h, wc_h)


def _rec_final_body(a_ref, s_ref, wg_ref, wc_ref, m_ref, wfc_ref, bfc_ref,
                    o_ref, state_ref, rs_ref, last_ref):
    t = pl.program_id(0)

    @pl.when(t == 0)
    def _():
        state_ref[...] = jnp.zeros_like(state_ref)
        last_ref[...] = jnp.zeros_like(last_ref)

    new_states = _gru_step(a_ref, s_ref[...], wg_ref[...], wc_ref[...],
                           state_ref, rs_ref)
    for bi in range(B):
        sl = slice(bi * H, (bi + 1) * H)
        state_ref[:, sl] = new_states[bi]
        mb = m_ref[0, 0, bi]
        last_ref[:, sl] = mb * new_states[bi] + (1.0 - mb) * last_ref[:, sl]

    @pl.when(t == T - 1)
    def _():
        wfc = wfc_ref[...]
        bfc = bfc_ref[0]
        for bi in range(B):
            sl = slice(bi * H, (bi + 1) * H)
            lg = _dot(jnp.maximum(last_ref[:, sl], 0.0), wfc) + bfc
            o_ref[bi:bi + 1, :] = jnp.max(lg, axis=0, keepdims=True)


def _recurrence_final(a, s, wg_h, wc_h, mask, wfc_pad, bfc_pad):
    return pl.pallas_call(
        _rec_final_body,
        grid=(T,),
        in_specs=[
            pl.BlockSpec((1, B, N, 3 * H), lambda t: (t, 0, 0, 0)),
            pl.BlockSpec((N, N), lambda t: (0, 0)),
            pl.BlockSpec((K * H, 2 * H), lambda t: (0, 0)),
            pl.BlockSpec((K * H, H), lambda t: (0, 0)),
            pl.BlockSpec((1, 1, B), lambda t: (t, 0, 0)),
            pl.BlockSpec((H, 128), lambda t: (0, 0)),
            pl.BlockSpec((1, 128), lambda t: (0, 0)),
        ],
        out_specs=pl.BlockSpec((B, 128), lambda t: (0, 0)),
        out_shape=jax.ShapeDtypeStruct((B, 128), F32),
        scratch_shapes=[
            pltpu.VMEM((N, B * H), F32),
            pltpu.VMEM((N, B * H), F32),
            pltpu.VMEM((N, B * H), F32),
        ],
    )(a, s, wg_h, wc_h, mask, wfc_pad, bfc_pad)


def _split_weight(w, din, dout):
    wr = w.reshape(din + H, K, dout)
    w_in = wr[:din].transpose(1, 0, 2).reshape(K * din, dout)
    w_h = wr[din:].transpose(1, 0, 2).reshape(K * H, dout)
    return w_in, w_h


def kernel(input_seq, seq_lengths, supports, Wg0, bg0, Wc0, bc0,
           Wg1, bg1, Wc1, bc1, Wfc, bfc):
    s = supports[0]

    wg0_in, wg0_h = _split_weight(Wg0, D, 2 * H)
    wc0_in, wc0_h = _split_weight(Wc0, D, H)
    wg1_in, wg1_h = _split_weight(Wg1, H, 2 * H)
    wc1_in, wc1_h = _split_weight(Wc1, H, H)
    w0_in = jnp.concatenate([wg0_in, wc0_in], axis=1)
    w1_in = jnp.concatenate([wg1_in, wc1_in], axis=1)
    bias0 = jnp.concatenate([bg0, bc0]).reshape(1, 3 * H)
    bias1 = jnp.concatenate([bg1, bc1]).reshape(1, 3 * H)

    idx = jnp.clip(seq_lengths - 1, 0, T - 1).astype(jnp.int32)
    mask = (jnp.arange(T, dtype=jnp.int32)[:, None, None]
            == idx[None, None, :]).astype(F32)

    wfc_pad = jnp.zeros((H, 128), F32).at[:, :C].set(Wfc)
    bfc_pad = jnp.zeros((1, 128), F32).at[0, :C].set(bfc)

    x0 = input_seq.transpose(1, 2, 0, 3).reshape(T, N, B * D)
    a0 = _precompute(x0, s, w0_in, bias0)
    out0 = _recurrence(a0, s, wg0_h, wc0_h)
    a1 = _precompute(out0, s, w1_in, bias1)
    logits_pad = _recurrence_final(a1, s, wg1_h, wc1_h, mask,
                                   wfc_pad, bfc_pad)
    return logits_pad[:, :C]


# speedup vs baseline: 15.8777x; 1.1278x over previous
import jax
import jax.numpy as jnp
from jax.experimental import pallas as pl
from jax.experimental.pallas import tpu as pltpu

N = 512
D = 128
H = 128
T = 16
B = 4
K = 3
C = 4
F32 = jnp.float32


def _dot(a, b):
    return jnp.dot(a, b, preferred_element_type=F32)


def _input_contrib(x_ref, s, w_in, bias):
    x0 = x_ref[0]
    x1 = _dot(s, x0)
    x2 = 2.0 * _dot(s, x1) - x0
    a_list = []
    for bi in range(B):
        sl = slice(bi * D, (bi + 1) * D)
        xc = jnp.concatenate([x0[:, sl], x1[:, sl], x2[:, sl]], axis=1)
        a_list.append(_dot(xc, w_in) + bias)
    return a_list


def _gru_step(a_list, s, wg, wc, state_ref, rs_ref):
    h0 = state_ref[...]
    h1 = _dot(s, h0)
    h2 = 2.0 * _dot(s, h1) - h0
    us = []
    for bi in range(B):
        sl = slice(bi * H, (bi + 1) * H)
        xc = jnp.concatenate([h0[:, sl], h1[:, sl], h2[:, sl]], axis=1)
        g = jax.nn.sigmoid(a_list[bi][:, : 2 * H] + _dot(xc, wg))
        r, u = g[:, :H], g[:, H:]
        rs_ref[:, sl] = r * h0[:, sl]
        us.append(u)
    rs0 = rs_ref[...]
    rs1 = _dot(s, rs0)
    rs2 = 2.0 * _dot(s, rs1) - rs0
    new_states = []
    for bi in range(B):
        sl = slice(bi * H, (bi + 1) * H)
        xc = jnp.concatenate([rs0[:, sl], rs1[:, sl], rs2[:, sl]], axis=1)
        c = jnp.tanh(a_list[bi][:, 2 * H:] + _dot(xc, wc))
        u = us[bi]
        new_states.append(u * h0[:, sl] + (1.0 - u) * c)
    return new_states


def _layer_body(x_ref, s_ref, win_ref, bin_ref, wg_ref, wc_ref,
                o_ref, state_ref, rs_ref):
    t = pl.program_id(0)

    @pl.when(t == 0)
    def _():
        state_ref[...] = jnp.zeros_like(state_ref)

    s = s_ref[...]
    a_list = _input_contrib(x_ref, s, win_ref[...], bin_ref[0])
    new_states = _gru_step(a_list, s, wg_ref[...], wc_ref[...],
                           state_ref, rs_ref)
    for bi in range(B):
        sl = slice(bi * H, (bi + 1) * H)
        state_ref[:, sl] = new_states[bi]
        o_ref[0, :, sl] = new_states[bi]


def _layer(x, s, w_in, bias_in, wg_h, wc_h):
    return pl.pallas_call(
        _layer_body,
        grid=(T,),
        in_specs=[
            pl.BlockSpec((1, N, B * D), lambda t: (t, 0, 0)),
            pl.BlockSpec((N, N), lambda t: (0, 0)),
            pl.BlockSpec((K * D, 3 * H), lambda t: (0, 0)),
            pl.BlockSpec((1, 3 * H), lambda t: (0, 0)),
            pl.BlockSpec((K * H, 2 * H), lambda t: (0, 0)),
            pl.BlockSpec((K * H, H), lambda t: (0, 0)),
        ],
        out_specs=pl.BlockSpec((1, N, B * H), lambda t: (t, 0, 0)),
        out_shape=jax.ShapeDtypeStruct((T, N, B * H), F32),
        scratch_shapes=[
            pltpu.VMEM((N, B * H), F32),
            pltpu.VMEM((N, B * H), F32),
        ],
    )(x, s, w_in, bias_in, wg_h, wc_h)


def _layer_final_body(x_ref, s_ref, win_ref, bin_ref, wg_ref, wc_ref,
                      m_ref, wfc_ref, bfc_ref,
                      o_ref, state_ref, rs_ref, last_ref):
    t = pl.program_id(0)

    @pl.when(t == 0)
    def _():
        state_ref[...] = jnp.zeros_like(state_ref)
        last_ref[...] = jnp.zeros_like(last_ref)

    s = s_ref[...]
    a_list = _input_contrib(x_ref, s, win_ref[...], bin_ref[0])
    new_states = _gru_step(a_list, s, wg_ref[...], wc_ref[...],
                           state_ref, rs_ref)
    for bi in range(B):
        sl = slice(bi * H, (bi + 1) * H)
        state_ref[:, sl] = new_states[bi]
        mb = m_ref[0, 0, bi]
        last_ref[:, sl] = mb * new_states[bi] + (1.0 - mb) * last_ref[:, sl]

    @pl.when(t == T - 1)
    def _():
        wfc = wfc_ref[...]
        bfc = bfc_ref[0]
        for bi in range(B):
            sl = slice(bi * H, (bi + 1) * H)
            lg = _dot(jnp.maximum(last_ref[:, sl], 0.0), wfc) + bfc
            o_ref[bi:bi + 1, :] = jnp.max(lg, axis=0, keepdims=True)


def _layer_final(x, s, w_in, bias_in, wg_h, wc_h, mask, wfc_pad, bfc_pad):
    return pl.pallas_call(
        _layer_final_body,
        grid=(T,),
        in_specs=[
            pl.BlockSpec((1, N, B * D), lambda t: (t, 0, 0)),
            pl.BlockSpec((N, N), lambda t: (0, 0)),
            pl.BlockSpec((K * D, 3 * H), lambda t: (0, 0)),
            pl.BlockSpec((1, 3 * H), lambda t: (0, 0)),
            pl.BlockSpec((K * H, 2 * H), lambda t: (0, 0)),
            pl.BlockSpec((K * H, H), lambda t: (0, 0)),
            pl.BlockSpec((1, 1, B), lambda t: (t, 0, 0)),
            pl.BlockSpec((H, 128), lambda t: (0, 0)),
            pl.BlockSpec((1, 128), lambda t: (0, 0)),
        ],
        out_specs=pl.BlockSpec((B, 128), lambda t: (0, 0)),
        out_shape=jax.ShapeDtypeStruct((B, 128), F32),
        scratch_shapes=[
            pltpu.VMEM((N, B * H), F32),
            pltpu.VMEM((N, B * H), F32),
            pltpu.VMEM((N, B * H), F32),
        ],
    )(x, s, w_in, bias_in, wg_h, wc_h, mask, wfc_pad, bfc_pad)


def _split_weight(w, din, dout):
    wr = w.reshape(din + H, K, dout)
    w_in = wr[:din].transpose(1, 0, 2).reshape(K * din, dout)
    w_h = wr[din:].transpose(1, 0, 2).reshape(K * H, dout)
    return w_in, w_h


def kernel(input_seq, seq_lengths, supports, Wg0, bg0, Wc0, bc0,
           Wg1, bg1, Wc1, bc1, Wfc, bfc):
    s = supports[0]

    wg0_in, wg0_h = _split_weight(Wg0, D, 2 * H)
    wc0_in, wc0_h = _split_weight(Wc0, D, H)
    wg1_in, wg1_h = _split_weight(Wg1, H, 2 * H)
    wc1_in, wc1_h = _split_weight(Wc1, H, H)
    w0_in = jnp.concatenate([wg0_in, wc0_in], axis=1)
    w1_in = jnp.concatenate([wg1_in, wc1_in], axis=1)
    bias0 = jnp.concatenate([bg0, bc0]).reshape(1, 3 * H)
    bias1 = jnp.concatenate([bg1, bc1]).reshape(1, 3 * H)

    idx = jnp.clip(seq_lengths - 1, 0, T - 1).astype(jnp.int32)
    mask = (jnp.arange(T, dtype=jnp.int32)[:, None, None]
            == idx[None, None, :]).astype(F32)

    wfc_pad = jnp.zeros((H, 128), F32).at[:, :C].set(Wfc)
    bfc_pad = jnp.zeros((1, 128), F32).at[0, :C].set(bfc)

    x0 = input_seq.transpose(1, 2, 0, 3).reshape(T, N, B * D)
    out0 = _layer(x0, s, w0_in, bias0, wg0_h, wc0_h)
    logits_pad = _layer_final(out0, s, w1_in, bias1, wg1_h, wc1_h, mask,
                              wfc_pad, bfc_pad)
    return logits_pad[:, :C]


# speedup vs baseline: 16.1096x; 1.0146x over previous
import jax
import jax.numpy as jnp
from jax.experimental import pallas as pl
from jax.experimental.pallas import tpu as pltpu

N = 512
D = 128
H = 128
T = 16
B = 4
K = 3
C = 4
F32 = jnp.float32


def _dot(a, b):
    return jnp.dot(a.astype(jnp.bfloat16), b.astype(jnp.bfloat16),
                   preferred_element_type=F32)


def _input_contrib(x_ref, s, w_in, bias):
    x0 = x_ref[0]
    x1 = _dot(s, x0)
    x2 = 2.0 * _dot(s, x1) - x0
    a_list = []
    for bi in range(B):
        sl = slice(bi * D, (bi + 1) * D)
        xc = jnp.concatenate([x0[:, sl], x1[:, sl], x2[:, sl]], axis=1)
        a_list.append(_dot(xc, w_in) + bias)
    return a_list


def _gru_step(a_list, s, wg, wc, state_ref, rs_ref):
    h0 = state_ref[...]
    h1 = _dot(s, h0)
    h2 = 2.0 * _dot(s, h1) - h0
    us = []
    for bi in range(B):
        sl = slice(bi * H, (bi + 1) * H)
        xc = jnp.concatenate([h0[:, sl], h1[:, sl], h2[:, sl]], axis=1)
        g = jax.nn.sigmoid(a_list[bi][:, : 2 * H] + _dot(xc, wg))
        r, u = g[:, :H], g[:, H:]
        rs_ref[:, sl] = r * h0[:, sl]
        us.append(u)
    rs0 = rs_ref[...]
    rs1 = _dot(s, rs0)
    rs2 = 2.0 * _dot(s, rs1) - rs0
    new_states = []
    for bi in range(B):
        sl = slice(bi * H, (bi + 1) * H)
        xc = jnp.concatenate([rs0[:, sl], rs1[:, sl], rs2[:, sl]], axis=1)
        c = jnp.tanh(a_list[bi][:, 2 * H:] + _dot(xc, wc))
        u = us[bi]
        new_states.append(u * h0[:, sl] + (1.0 - u) * c)
    return new_states


def _layer_body(x_ref, s_ref, win_ref, bin_ref, wg_ref, wc_ref,
                o_ref, state_ref, rs_ref):
    t = pl.program_id(0)

    @pl.when(t == 0)
    def _():
        state_ref[...] = jnp.zeros_like(state_ref)

    s = s_ref[...]
    a_list = _input_contrib(x_ref, s, win_ref[...], bin_ref[0])
    new_states = _gru_step(a_list, s, wg_ref[...], wc_ref[...],
                           state_ref, rs_ref)
    for bi in range(B):
        sl = slice(bi * H, (bi + 1) * H)
        state_ref[:, sl] = new_states[bi]
        o_ref[0, :, sl] = new_states[bi]


def _layer(x, s, w_in, bias_in, wg_h, wc_h):
    return pl.pallas_call(
        _layer_body,
        grid=(T,),
        in_specs=[
            pl.BlockSpec((1, N, B * D), lambda t: (t, 0, 0)),
            pl.BlockSpec((N, N), lambda t: (0, 0)),
            pl.BlockSpec((K * D, 3 * H), lambda t: (0, 0)),
            pl.BlockSpec((1, 3 * H), lambda t: (0, 0)),
            pl.BlockSpec((K * H, 2 * H), lambda t: (0, 0)),
            pl.BlockSpec((K * H, H), lambda t: (0, 0)),
        ],
        out_specs=pl.BlockSpec((1, N, B * H), lambda t: (t, 0, 0)),
        out_shape=jax.ShapeDtypeStruct((T, N, B * H), F32),
        scratch_shapes=[
            pltpu.VMEM((N, B * H), F32),
            pltpu.VMEM((N, B * H), F32),
        ],
    )(x, s, w_in, bias_in, wg_h, wc_h)


def _layer_final_body(x_ref, s_ref, win_ref, bin_ref, wg_ref, wc_ref,
                      m_ref, wfc_ref, bfc_ref,
                      o_ref, state_ref, rs_ref, last_ref):
    t = pl.program_id(0)

    @pl.when(t == 0)
    def _():
        state_ref[...] = jnp.zeros_like(state_ref)
        last_ref[...] = jnp.zeros_like(last_ref)

    s = s_ref[...]
    a_list = _input_contrib(x_ref, s, win_ref[...], bin_ref[0])
    new_states = _gru_step(a_list, s, wg_ref[...], wc_ref[...],
                           state_ref, rs_ref)
    for bi in range(B):
        sl = slice(bi * H, (bi + 1) * H)
        state_ref[:, sl] = new_states[bi]
        mb = m_ref[0, 0, bi]
        last_ref[:, sl] = mb * new_states[bi] + (1.0 - mb) * last_ref[:, sl]

    @pl.when(t == T - 1)
    def _():
        wfc = wfc_ref[...]
        bfc = bfc_ref[0]
        for bi in range(B):
            sl = slice(bi * H, (bi + 1) * H)
            lg = _dot(jnp.maximum(last_ref[:, sl], 0.0), wfc) + bfc
            o_ref[bi:bi + 1, :] = jnp.max(lg, axis=0, keepdims=True)


def _layer_final(x, s, w_in, bias_in, wg_h, wc_h, mask, wfc_pad, bfc_pad):
    return pl.pallas_call(
        _layer_final_body,
        grid=(T,),
        in_specs=[
            pl.BlockSpec((1, N, B * D), lambda t: (t, 0, 0)),
            pl.BlockSpec((N, N), lambda t: (0, 0)),
            pl.BlockSpec((K * D, 3 * H), lambda t: (0, 0)),
            pl.BlockSpec((1, 3 * H), lambda t: (0, 0)),
            pl.BlockSpec((K * H, 2 * H), lambda t: (0, 0)),
            pl.BlockSpec((K * H, H), lambda t: (0, 0)),
            pl.BlockSpec((1, 1, B), lambda t: (t, 0, 0)),
            pl.BlockSpec((H, 128), lambda t: (0, 0)),
            pl.BlockSpec((1, 128), lambda t: (0, 0)),
        ],
        out_specs=pl.BlockSpec((B, 128), lambda t: (0, 0)),
        out_shape=jax.ShapeDtypeStruct((B, 128), F32),
        scratch_shapes=[
            pltpu.VMEM((N, B * H), F32),
            pltpu.VMEM((N, B * H), F32),
            pltpu.VMEM((N, B * H), F32),
        ],
    )(x, s, w_in, bias_in, wg_h, wc_h, mask, wfc_pad, bfc_pad)


def _split_weight(w, din, dout):
    wr = w.reshape(din + H, K, dout)
    w_in = wr[:din].transpose(1, 0, 2).reshape(K * din, dout)
    w_h = wr[din:].transpose(1, 0, 2).reshape(K * H, dout)
    return w_in, w_h


def kernel(input_seq, seq_lengths, supports, Wg0, bg0, Wc0, bc0,
           Wg1, bg1, Wc1, bc1, Wfc, bfc):
    s = supports[0]

    wg0_in, wg0_h = _split_weight(Wg0, D, 2 * H)
    wc0_in, wc0_h = _split_weight(Wc0, D, H)
    wg1_in, wg1_h = _split_weight(Wg1, H, 2 * H)
    wc1_in, wc1_h = _split_weight(Wc1, H, H)
    w0_in = jnp.concatenate([wg0_in, wc0_in], axis=1)
    w1_in = jnp.concatenate([wg1_in, wc1_in], axis=1)
    bias0 = jnp.concatenate([bg0, bc0]).reshape(1, 3 * H)
    bias1 = jnp.concatenate([bg1, bc1]).reshape(1, 3 * H)

    idx = jnp.clip(seq_lengths - 1, 0, T - 1).astype(jnp.int32)
    mask = (jnp.arange(T, dtype=jnp.int32)[:, None, None]
            == idx[None, None, :]).astype(F32)

    wfc_pad = jnp.zeros((H, 128), F32).at[:, :C].set(Wfc)
    bfc_pad = jnp.zeros((1, 128), F32).at[0, :C].set(bfc)

    x0 = input_seq.transpose(1, 2, 0, 3).reshape(T, N, B * D)
    out0 = _layer(x0, s, w0_in, bias0, wg0_h, wc0_h)
    logits_pad = _layer_final(out0, s, w1_in, bias1, wg1_h, wc1_h, mask,
                              wfc_pad, bfc_pad)
    return logits_pad[:, :C]
